# baseline (speedup 1.0000x reference)
import functools

import jax
import jax.numpy as jnp
from jax import lax
from jax.experimental import pallas as pl
from jax.experimental.pallas import tpu as pltpu
from jax.experimental.pallas import tpu_sc as plsc

NUM_CORES = 2
NUM_SUBCORES = 16
NUM_WORKERS = NUM_CORES * NUM_SUBCORES
CHUNK = 40
NBUF = 8
UNROLL = 8
LANES = 16


def _gather_kernel(batch, seq, d, d_pad):
    n_flat = batch * seq
    n_chunks = n_flat // CHUNK // NUM_WORKERS
    n_groups = n_chunks // NBUF - 1
    seq_chunks = seq // CHUNK
    mesh = plsc.VectorSubcoreMesh(core_axis_name="c", subcore_axis_name="s")

    spans = [0, 16, 32, d - LANES]

    @functools.partial(
        pl.kernel,
        mesh=mesh,
        compiler_params=pltpu.CompilerParams(use_tc_tiling_on_sc=False),
        out_type=jax.ShapeDtypeStruct((batch, seq, d), jnp.float32),
        scratch_types=[
            pltpu.VMEM((n_chunks, CHUNK), jnp.int32),
            pltpu.VMEM((NBUF, CHUNK, d_pad), jnp.float32),
            pltpu.VMEM((NBUF, CHUNK, d), jnp.float32),
            pltpu.SemaphoreType.DMA((NBUF,)),
            pltpu.SemaphoreType.DMA((NBUF,)),
        ],
    )
    def k(idx_hbm, table_hbm, out_hbm, idx_v, rows_v, out_v, gsem, wsem):
        wid = lax.axis_index("s") * NUM_CORES + lax.axis_index("c")
        base = wid * n_chunks

        def gather(c, b):
            pltpu.async_copy(
                table_hbm.at[idx_v.at[c]], rows_v.at[b], gsem.at[b]
            )

        def compact(b):
            def row_body(q, carry):
                r0 = q * UNROLL
                for u in range(UNROLL):
                    for a in spans:
                        out_v[b, r0 + u, pl.ds(a, LANES)] = (
                            rows_v[b, r0 + u, pl.ds(a, LANES)]
                        )
                return carry

            lax.fori_loop(0, CHUNK // UNROLL, row_body, 0)

        def write(c, b):
            gc = base + c
            bi = gc // seq_chunks
            s0 = (gc % seq_chunks) * CHUNK
            pltpu.async_copy(
                out_v.at[b], out_hbm.at[bi, pl.ds(s0, CHUNK)], wsem.at[b]
            )

        def wait_gather(b):
            pltpu.make_async_copy(
                table_hbm.at[pl.ds(0, CHUNK)], rows_v.at[b], gsem.at[b]
            ).wait()

        def wait_write(b):
            pltpu.make_async_copy(
                out_v.at[b], out_hbm.at[0, pl.ds(0, CHUNK)], wsem.at[b]
            ).wait()

        pltpu.sync_copy(idx_hbm.at[pl.ds(wid * n_chunks, n_chunks)], idx_v)

        for b in range(NBUF):
            gather(b, b)

        def body(g, carry):
            c0 = g * NBUF
            for b in range(NBUF):
                c = c0 + b
                wait_gather(b)
                compact(b)
                write(c, b)
                wait_write(b)
                gather(c + NBUF, b)
            return carry

        lax.fori_loop(0, n_groups, body, 0)

        for b in range(NBUF):
            c = n_groups * NBUF + b
            wait_gather(b)
            compact(b)
            write(c, b)
        for b in range(NBUF):
            wait_write(b)

    return k


def kernel(indices, table):
    batch, seq = indices.shape
    vocab, d = table.shape
    d_pad = (d + 7) // 8 * 8
    n_flat = batch * seq
    idx2d = indices.reshape(n_flat // CHUNK, CHUNK).astype(jnp.int32)
    table_pad = jnp.pad(table, ((0, 0), (0, d_pad - d)))
    return _gather_kernel(batch, seq, d, d_pad)(idx2d, table_pad)


# speedup vs baseline: 1.2636x; 1.2636x over previous
import functools

import jax
import jax.numpy as jnp
from jax import lax
from jax.experimental import pallas as pl
from jax.experimental.pallas import tpu as pltpu
from jax.experimental.pallas import tpu_sc as plsc

NUM_CORES = 2
NUM_SUBCORES = 16
NUM_WORKERS = NUM_CORES * NUM_SUBCORES
CHUNK = 40
NBUF = 8
NHALF = 2
UNROLL = 8
LANES = 16


def _gather_kernel(batch, seq, d, d_pad):
    n_flat = batch * seq
    n_chunks = n_flat // CHUNK // NUM_WORKERS
    h_chunks = n_chunks // NHALF
    n_groups = h_chunks // NBUF - 1
    seq_chunks = seq // CHUNK
    mesh = plsc.VectorSubcoreMesh(core_axis_name="c", subcore_axis_name="s")

    spans = [0, 16, 32, d - LANES]

    @functools.partial(
        pl.kernel,
        mesh=mesh,
        compiler_params=pltpu.CompilerParams(use_tc_tiling_on_sc=True),
        out_type=jax.ShapeDtypeStruct((batch, seq, d), jnp.float32),
        scratch_types=[
            pltpu.VMEM((h_chunks, CHUNK), jnp.int32),
            pltpu.VMEM((NBUF, CHUNK, d_pad), jnp.float32),
            pltpu.VMEM((NBUF, CHUNK, d), jnp.float32),
            pltpu.SemaphoreType.DMA((NBUF,)),
            pltpu.SemaphoreType.DMA((NBUF,)),
        ],
    )
    def k(idx_hbm, table_hbm, out_hbm, idx_v, rows_v, out_v, gsem, wsem):
        wid = lax.axis_index("s") * NUM_CORES + lax.axis_index("c")
        base = wid * n_chunks

        def gather(c, b):
            pltpu.async_copy(
                table_hbm.at[idx_v.at[c]], rows_v.at[b], gsem.at[b]
            )

        def compact(b):
            def row_body(q, carry):
                r0 = q * UNROLL
                for u in range(UNROLL):
                    for a in spans:
                        out_v[b, r0 + u, pl.ds(a, LANES)] = (
                            rows_v[b, r0 + u, pl.ds(a, LANES)]
                        )
                return carry

            lax.fori_loop(0, CHUNK // UNROLL, row_body, 0)

        def write(gc, b):
            bi = gc // seq_chunks
            s0 = (gc % seq_chunks) * CHUNK
            pltpu.async_copy(
                out_v.at[b], out_hbm.at[bi, pl.ds(s0, CHUNK)], wsem.at[b]
            )

        def wait_gather(b):
            pltpu.make_async_copy(
                table_hbm.at[pl.ds(0, CHUNK)], rows_v.at[b], gsem.at[b]
            ).wait()

        def wait_write(b):
            pltpu.make_async_copy(
                out_v.at[b], out_hbm.at[0, pl.ds(0, CHUNK)], wsem.at[b]
            ).wait()

        for half in range(NHALF):
            h0 = half * h_chunks
            pltpu.sync_copy(
                idx_hbm.at[pl.ds(base + h0, h_chunks)], idx_v
            )

            for b in range(NBUF):
                gather(b, b)

            def body(g, carry):
                c0 = g * NBUF
                for b in range(NBUF):
                    c = c0 + b
                    wait_gather(b)
                    compact(b)
                    write(base + h0 + c, b)
                    wait_write(b)
                    gather(c + NBUF, b)
                return carry

            lax.fori_loop(0, n_groups, body, 0)

            for b in range(NBUF):
                c = n_groups * NBUF + b
                wait_gather(b)
                compact(b)
                write(base + h0 + c, b)
            for b in range(NBUF):
                wait_write(b)

    return k


def kernel(indices, table):
    batch, seq = indices.shape
    vocab, d = table.shape
    d_pad = 128
    n_flat = batch * seq
    idx2d = indices.reshape(n_flat // CHUNK, CHUNK).astype(jnp.int32)
    table_pad = jnp.pad(table, ((0, 0), (0, d_pad - d)))
    return _gather_kernel(batch, seq, d, d_pad)(idx2d, table_pad)


# speedup vs baseline: 1.2648x; 1.0009x over previous
import functools

import jax
import jax.numpy as jnp
from jax import lax
from jax.experimental import pallas as pl
from jax.experimental.pallas import tpu as pltpu
from jax.experimental.pallas import tpu_sc as plsc

NUM_CORES = 2
NUM_SUBCORES = 16
NUM_WORKERS = NUM_CORES * NUM_SUBCORES
CHUNK = 40
NBUF = 8
NHALF = 2
UNROLL = 8
LANES = 16


def _gather_kernel(batch, seq, d, d_pad):
    n_flat = batch * seq
    n_chunks = n_flat // CHUNK // NUM_WORKERS
    h_chunks = n_chunks // NHALF
    n_groups = h_chunks // NBUF - 1
    seq_chunks = seq // CHUNK
    mesh = plsc.VectorSubcoreMesh(core_axis_name="c", subcore_axis_name="s")

    spans = [0, 16, 32, d - LANES]

    @functools.partial(
        pl.kernel,
        mesh=mesh,
        compiler_params=pltpu.CompilerParams(use_tc_tiling_on_sc=True),
        out_type=jax.ShapeDtypeStruct((batch, seq, d), jnp.float32),
        scratch_types=[
            pltpu.VMEM((h_chunks, CHUNK), jnp.int32),
            pltpu.VMEM((NBUF, CHUNK, d_pad), jnp.float32),
            pltpu.VMEM((NBUF, CHUNK, d), jnp.float32),
            pltpu.SemaphoreType.DMA((NBUF,)),
            pltpu.SemaphoreType.DMA((NBUF,)),
        ],
    )
    def k(idx_hbm, table_hbm, out_hbm, idx_v, rows_v, out_v, gsem, wsem):
        wid = lax.axis_index("s") * NUM_CORES + lax.axis_index("c")
        base = wid * n_chunks

        def gather(c, b):
            pltpu.async_copy(
                table_hbm.at[idx_v.at[c]], rows_v.at[b], gsem.at[b]
            )

        def compact(b):
            def row_body(q, carry):
                r0 = q * UNROLL
                for u in range(UNROLL):
                    for a in spans:
                        out_v[b, r0 + u, pl.ds(a, LANES)] = (
                            rows_v[b, r0 + u, pl.ds(a, LANES)]
                        )
                return carry

            lax.fori_loop(0, CHUNK // UNROLL, row_body, 0)

        def write(gc, b):
            bi = gc // seq_chunks
            s0 = (gc % seq_chunks) * CHUNK
            pltpu.async_copy(
                out_v.at[b], out_hbm.at[bi, pl.ds(s0, CHUNK)], wsem.at[b]
            )

        def wait_gather(b):
            pltpu.make_async_copy(
                table_hbm.at[pl.ds(0, CHUNK)], rows_v.at[b], gsem.at[b]
            ).wait()

        def wait_write(b):
            pltpu.make_async_copy(
                out_v.at[b], out_hbm.at[0, pl.ds(0, CHUNK)], wsem.at[b]
            ).wait()

        for half in range(NHALF):
            h0 = half * h_chunks
            pltpu.sync_copy(
                idx_hbm.at[pl.ds(base + h0, h_chunks)], idx_v
            )

            for b in range(NBUF):
                gather(b, b)

            for b in range(NBUF):
                wait_gather(b)
                compact(b)
                write(base + h0 + b, b)
                gather(b + NBUF, b)

            def body(g, carry):
                c0 = g * NBUF
                for b in range(NBUF):
                    c = c0 + b
                    wait_gather(b)
                    wait_write(b)
                    compact(b)
                    write(base + h0 + c, b)
                    gather(c + NBUF, b)
                return carry

            lax.fori_loop(1, n_groups, body, 0)

            for b in range(NBUF):
                c = n_groups * NBUF + b
                wait_gather(b)
                wait_write(b)
                compact(b)
                write(base + h0 + c, b)
            for b in range(NBUF):
                wait_write(b)

    return k


def kernel(indices, table):
    batch, seq = indices.shape
    vocab, d = table.shape
    d_pad = 128
    n_flat = batch * seq
    idx2d = indices.reshape(n_flat // CHUNK, CHUNK).astype(jnp.int32)
    table_pad = jnp.pad(table, ((0, 0), (0, d_pad - d)))
    return _gather_kernel(batch, seq, d, d_pad)(idx2d, table_pad)
